# baseline (speedup 1.0000x reference)
import functools

import jax
import jax.numpy as jnp
from jax.experimental import pallas as pl
from jax.experimental.pallas import tpu as pltpu


def _body(x_ref, m_ref, o_ref, acc_s, acc_c):
    ph = pl.program_id(1)
    l = pl.program_id(2)

    @pl.when(ph == 0)
    def _accumulate():
        @pl.when(l == 0)
        def _init():
            acc_s[...] = jnp.zeros_like(acc_s)
            acc_c[...] = jnp.zeros_like(acc_c)

        m = m_ref[...].astype(jnp.float32)
        acc_s[...] += jnp.sum(x_ref[...] * m, axis=1)
        acc_c[...] += jnp.sum(m, axis=1)

    @pl.when(ph == 1)
    def _write():
        cnt = acc_c[...]
        mean = jnp.where(cnt > 0, acc_s[...] / jnp.maximum(cnt, 1.0), 0.0)
        o_ref[...] = jnp.broadcast_to(mean[:, None], o_ref.shape)


def kernel(x, mask):
    B, L, K, C = x.shape
    LB = 512
    nlb = L // LB

    def in_map(b, ph, l):
        return (b, l * (1 - ph) + (nlb - 1) * ph, 0, 0)

    def out_map(b, ph, l):
        return (b, l * ph, 0, 0)

    return pl.pallas_call(
        _body,
        out_shape=jax.ShapeDtypeStruct((B, L, K, C), x.dtype),
        grid=(B, 2, nlb),
        in_specs=[
            pl.BlockSpec((1, LB, K, C), in_map),
            pl.BlockSpec((1, LB, K, C), in_map),
        ],
        out_specs=pl.BlockSpec((1, LB, K, C), out_map),
        scratch_shapes=[
            pltpu.VMEM((1, K, C), jnp.float32),
            pltpu.VMEM((1, K, C), jnp.float32),
        ],
        compiler_params=pltpu.CompilerParams(
            dimension_semantics=("parallel", "arbitrary", "arbitrary"),
            vmem_limit_bytes=61 * 1024 * 1024,
        ),
        name="masked_mean_bcast",
    )(x, mask)


# speedup vs baseline: 7.2996x; 7.2996x over previous
import jax
import jax.numpy as jnp
from jax.experimental import pallas as pl
from jax.experimental.pallas import tpu as pltpu


def _body(x_ref, m_ref, o_ref):
    m = m_ref[...].astype(jnp.float32)
    s = jnp.sum(x_ref[...] * m, axis=3, keepdims=True)
    cnt = jnp.sum(m, axis=3, keepdims=True)
    mean = jnp.where(cnt > 0, s / jnp.maximum(cnt, 1.0), 0.0)
    o_ref[...] = jnp.broadcast_to(mean, o_ref.shape)


def kernel(x, mask):
    B, L, K, C = x.shape
    xt = jnp.transpose(x, (0, 2, 3, 1))
    mt = jnp.transpose(mask, (0, 2, 3, 1))

    KB = 8
    grid = (B, K // KB)
    spec = pl.BlockSpec((1, KB, C, L), lambda b, k: (b, k, 0, 0))

    out = pl.pallas_call(
        _body,
        out_shape=jax.ShapeDtypeStruct((B, K, C, L), x.dtype),
        grid=grid,
        in_specs=[spec, spec],
        out_specs=spec,
        compiler_params=pltpu.CompilerParams(
            dimension_semantics=("parallel", "arbitrary"),
            vmem_limit_bytes=61 * 1024 * 1024,
        ),
        name="masked_mean_bcast",
    )(xt, mt)
    return jnp.transpose(out, (0, 3, 1, 2))
